# baseline (speedup 1.0000x reference)
import functools

import jax
import jax.numpy as jnp
from jax import lax
from jax.experimental import pallas as pl
from jax.experimental.pallas import tpu as pltpu
from jax.experimental.pallas import tpu_sc as plsc

F32 = jnp.float32
I32 = jnp.int32


def _sc_gather(table, idx, k):
    n = idx.shape[0]
    d = table.shape[1]
    info = plsc.get_sparse_core_info()
    nc, ns = info.num_cores, info.num_subcores
    nw = nc * ns
    per_w = n // nw
    nblk = per_w // k

    mesh = plsc.VectorSubcoreMesh(core_axis_name="c", subcore_axis_name="s")

    @functools.partial(
        pl.kernel,
        out_type=jax.ShapeDtypeStruct((n, d), F32),
        mesh=mesh,
        scratch_types=[
            pltpu.VMEM((k,), I32),
            pltpu.VMEM((k, d), F32),
            pltpu.SemaphoreType.DMA,
        ],
    )
    def gk(table_hbm, idx_hbm, out_hbm, idx_v, rows_v, sem):
        wid = lax.axis_index("s") * nc + lax.axis_index("c")
        base = wid * per_w

        def body(j, carry):
            off = base + j * k
            pltpu.sync_copy(idx_hbm.at[pl.ds(off, k)], idx_v)
            pltpu.async_copy(table_hbm.at[idx_v], rows_v, sem).wait()
            pltpu.sync_copy(rows_v, out_hbm.at[pl.ds(off, k)])
            return carry

        lax.fori_loop(0, nblk, body, 0)

    return gk(table, idx)


def _sc_scatter_add(ext0, ext1, dst_idx, nd, k):
    e, dext = ext0.shape
    info = plsc.get_sparse_core_info()
    nc, ns = info.num_cores, info.num_subcores
    per_tile = e // ns
    nblk = per_tile // k
    rows_per_tile = ((nd + 8 * ns - 1) // (8 * ns)) * 8
    nd_pad = rows_per_tile * ns

    zeros = jnp.zeros((rows_per_tile, dext), F32)
    mesh = plsc.VectorSubcoreMesh(core_axis_name="c", subcore_axis_name="s")

    @functools.partial(
        pl.kernel,
        out_type=(
            jax.ShapeDtypeStruct((nd_pad, dext), F32),
            jax.ShapeDtypeStruct((nd_pad, dext), F32),
        ),
        mesh=mesh,
        scratch_types=[
            pltpu.VMEM((k,), I32),
            pltpu.VMEM((k, dext), F32),
            pltpu.SemaphoreType.DMA,
            pltpu.VMEM_SHARED((nd_pad, dext), F32),
        ],
    )
    def sk(e0_hbm, e1_hbm, dst_hbm, z_hbm, out0, out1, idx_v, rows_v, sem, acc):
        cid = lax.axis_index("c")
        sid = lax.axis_index("s")
        my_rows = pl.ds(sid * rows_per_tile, rows_per_tile)
        pltpu.sync_copy(z_hbm, acc.at[my_rows])
        plsc.subcore_barrier()

        base = sid * per_tile

        def body(ext_hbm):
            def step(j, carry):
                off = base + j * k
                pltpu.sync_copy(dst_hbm.at[pl.ds(off, k)], idx_v)
                pltpu.sync_copy(ext_hbm.at[pl.ds(off, k)], rows_v)
                pltpu.sync_copy(rows_v, acc.at[idx_v], add=True)
                return carry

            lax.fori_loop(0, nblk, step, 0)

        @pl.when(cid == 0)
        def _():
            body(e0_hbm)

        @pl.when(cid == 1)
        def _():
            body(e1_hbm)

        plsc.subcore_barrier()

        @pl.when(cid == 0)
        def _():
            pltpu.sync_copy(acc.at[my_rows], out0.at[my_rows])

        @pl.when(cid == 1)
        def _():
            pltpu.sync_copy(acc.at[my_rows], out1.at[my_rows])

    return sk(ext0, ext1, dst_idx, zeros)


def _tc_qn(src_data, wqn_t, bqp, nd, b):
    def body(q_ref, w_ref, b_ref, o_ref):
        o_ref[...] = (
            jnp.dot(q_ref[...], w_ref[...], preferred_element_type=F32) + b_ref[...]
        )

    return pl.pallas_call(
        body,
        grid=(nd // b,),
        in_specs=[
            pl.BlockSpec((b, 128), lambda i: (i, 0)),
            pl.BlockSpec((128, 128), lambda i: (0, 0)),
            pl.BlockSpec((1, 128), lambda i: (0, 0)),
        ],
        out_specs=pl.BlockSpec((b, 128), lambda i: (i, 0)),
        out_shape=jax.ShapeDtypeStruct((nd, 128), F32),
        compiler_params=pltpu.CompilerParams(
            dimension_semantics=("parallel",)
        ),
    )(src_data, wqn_t, bqp)


def _tc_edge_pass(src_data, qe, edge_feat, dt2, wk_parts, wv_parts, twp, tbp,
                  e, nd, b, dh, dext):
    wkn_t, wke_t, wkt_t, bk2 = wk_parts
    wvn_t, wve_t, wvt_t, bv2 = wv_parts
    row_off = nd // b
    de = edge_feat.shape[1]

    bf16 = jnp.bfloat16

    def body(kv_ref, qe_ref, ef_ref, dt_ref,
             wkn, wke, wkt, bkr, wvn, wve, wvt, bvr, twr, tbr, o0_ref, o1_ref):
        tf = jnp.cos(dt_ref[...] * twr[...] + tbr[...]).astype(bf16)
        kv = kv_ref[...].astype(bf16)
        ef = ef_ref[...].astype(bf16)
        k_mat = (
            jnp.dot(kv, wkn[...], preferred_element_type=F32)
            + jnp.dot(ef, wke[...], preferred_element_type=F32)
            + jnp.dot(tf, wkt[...], preferred_element_type=F32)
            + bkr[...]
        )
        v_mat = (
            jnp.dot(kv, wvn[...], preferred_element_type=F32)
            + jnp.dot(ef, wve[...], preferred_element_type=F32)
            + jnp.dot(tf, wvt[...], preferred_element_type=F32)
            + bvr[...]
        )
        s = qe_ref[...] * k_mat
        d0 = jnp.sum(s[:, :dh], axis=1, keepdims=True)
        d1 = jnp.sum(s[:, dh:], axis=1, keepdims=True)
        l0 = jnp.where(d0 >= 0, d0, 0.2 * d0)
        l1 = jnp.where(d1 >= 0, d1, 0.2 * d1)
        ex0 = jnp.exp(jnp.minimum(l0, 75.0))
        ex1 = jnp.exp(jnp.minimum(l1, 75.0))
        z = jnp.zeros((b, dext - dh - 1), F32)
        o0_ref[...] = jnp.concatenate(
            [v_mat[:, :dh] * jnp.broadcast_to(ex0, (b, dh)), ex0, z], axis=1
        )
        o1_ref[...] = jnp.concatenate(
            [v_mat[:, dh:] * jnp.broadcast_to(ex1, (b, dh)), ex1, z], axis=1
        )

    wspec = lambda shape: pl.BlockSpec(shape, lambda i: (0, 0))
    return pl.pallas_call(
        body,
        grid=(e // b,),
        in_specs=[
            pl.BlockSpec((b, 128), lambda i: (i + row_off, 0)),
            pl.BlockSpec((b, 128), lambda i: (i, 0)),
            pl.BlockSpec((b, de), lambda i: (i, 0)),
            pl.BlockSpec((b, 1), lambda i: (i, 0)),
            wspec((128, 128)), wspec((de, 128)), wspec((128, 128)), wspec((1, 128)),
            wspec((128, 128)), wspec((de, 128)), wspec((128, 128)), wspec((1, 128)),
            wspec((1, 128)), wspec((1, 128)),
        ],
        out_specs=(
            pl.BlockSpec((b, dext), lambda i: (i, 0)),
            pl.BlockSpec((b, dext), lambda i: (i, 0)),
        ),
        out_shape=(
            jax.ShapeDtypeStruct((e, dext), F32),
            jax.ShapeDtypeStruct((e, dext), F32),
        ),
        compiler_params=pltpu.CompilerParams(
            dimension_semantics=("parallel",)
        ),
    )(src_data, qe, edge_feat, dt2,
      wkn_t, wke_t, wkt_t, bk2, wvn_t, wve_t, wvt_t, bv2, twp, tbp)


def _tc_final(p0, p1, src_data, wod_t, wos_t, bo2, g2, b2, nd, b, dh, dext):
    def body(p0_ref, p1_ref, q_ref, wod, wos, bor, gr, br, o_ref):
        p0 = p0_ref[...]
        p1 = p1_ref[...]
        den0 = jnp.maximum(p0[:, dh : dh + 1], 1e-16)
        den1 = jnp.maximum(p1[:, dh : dh + 1], 1e-16)
        dst_h = jnp.concatenate(
            [p0[:, :dh] / jnp.broadcast_to(den0, (b, dh)),
             p1[:, :dh] / jnp.broadcast_to(den1, (b, dh))], axis=1
        )
        r = (
            jnp.dot(dst_h, wod[...], preferred_element_type=F32)
            + jnp.dot(q_ref[...], wos[...], preferred_element_type=F32)
            + bor[...]
        )
        r = jnp.maximum(r, 0.0)
        mu = jnp.mean(r, axis=1, keepdims=True)
        var = jnp.mean((r - mu) ** 2, axis=1, keepdims=True)
        o_ref[...] = (r - mu) / jnp.sqrt(var + 1e-5) * gr[...] + br[...]

    wspec = lambda shape: pl.BlockSpec(shape, lambda i: (0, 0))
    return pl.pallas_call(
        body,
        grid=(nd // b,),
        in_specs=[
            pl.BlockSpec((b, dext), lambda i: (i, 0)),
            pl.BlockSpec((b, dext), lambda i: (i, 0)),
            pl.BlockSpec((b, 128), lambda i: (i, 0)),
            wspec((128, 128)), wspec((128, 128)),
            wspec((1, 128)), wspec((1, 128)), wspec((1, 128)),
        ],
        out_specs=pl.BlockSpec((b, 128), lambda i: (i, 0)),
        out_shape=jax.ShapeDtypeStruct((nd, 128), F32),
        compiler_params=pltpu.CompilerParams(
            dimension_semantics=("parallel",)
        ),
    )(p0, p1, src_data, wod_t, wos_t, bo2, g2, b2)


def kernel(h, src_idx, edge_dt, edge_feat, edge_dst, num_dst, time_w, time_b,
           Wq, bq, Wk, bk, Wv, bv, Wo, bo, ln_g, ln_b):
    e = edge_dst.shape[0]
    n_src, dn = h.shape
    nd = n_src - e
    dt_dim = time_w.shape[0]
    dout = Wq.shape[0]
    dh = dout // 2
    de = edge_feat.shape[1]
    dext = 128
    b = 400
    k = 80
    nw = 32

    wqn_t = Wq[:, :dn].T
    bqp = (bq + jnp.cos(time_b) @ Wq[:, dn:].T).reshape(1, dout)
    bf16 = jnp.bfloat16
    wkn_t = Wk[:, :dn].T.astype(bf16)
    wke_t = Wk[:, dn : dn + de].T.astype(bf16)
    wkt_t = jnp.pad(Wk[:, dn + de :].T, ((0, 128 - dt_dim), (0, 0))).astype(bf16)
    wvn_t = Wv[:, :dn].T.astype(bf16)
    wve_t = Wv[:, dn : dn + de].T.astype(bf16)
    wvt_t = jnp.pad(Wv[:, dn + de :].T, ((0, 128 - dt_dim), (0, 0))).astype(bf16)
    twp = jnp.pad(time_w[:, 0], (0, 128 - dt_dim)).reshape(1, 128)
    tbp = jnp.pad(time_b, (0, 128 - dt_dim)).reshape(1, 128)
    wod_t = Wo[:, :dout].T
    wos_t = Wo[:, dout:].T
    bo2 = bo.reshape(1, dout)
    g2 = ln_g.reshape(1, dout)
    b2 = ln_b.reshape(1, dout)
    bk2 = bk.reshape(1, dout)
    bv2 = bv.reshape(1, dout)

    chunk = nw * k
    n_pad = ((n_src + chunk - 1) // chunk) * chunk
    si = jnp.pad(src_idx.astype(I32), (0, n_pad - n_src))
    src_data = _sc_gather(h, si, k)

    qn = _tc_qn(src_data, wqn_t, bqp, nd, b)

    dst_i = edge_dst.astype(I32)
    qe = _sc_gather(qn, dst_i, k)

    dt2 = edge_dt.reshape(e, 1)
    ext0, ext1 = _tc_edge_pass(src_data, qe, edge_feat, dt2,
                               (wkn_t, wke_t, wkt_t, bk2), (wvn_t, wve_t, wvt_t, bv2),
                               twp, tbp, e, nd, b, dh, dext)

    p0, p1 = _sc_scatter_add(ext0, ext1, dst_i, nd, k)

    return _tc_final(p0, p1, src_data, wod_t, wos_t, bo2, g2, b2, nd, b, dh, dext)


# speedup vs baseline: 1.1897x; 1.1897x over previous
import functools

import jax
import jax.numpy as jnp
from jax import lax
from jax.experimental import pallas as pl
from jax.experimental.pallas import tpu as pltpu
from jax.experimental.pallas import tpu_sc as plsc

F32 = jnp.float32
I32 = jnp.int32


def _sc_gather(table, idx, k):
    n = idx.shape[0]
    d = table.shape[1]
    info = plsc.get_sparse_core_info()
    nc, ns = info.num_cores, info.num_subcores
    nw = nc * ns
    per_w = n // nw
    nblk = per_w // k

    mesh = plsc.VectorSubcoreMesh(core_axis_name="c", subcore_axis_name="s")

    @functools.partial(
        pl.kernel,
        out_type=jax.ShapeDtypeStruct((n, d), F32),
        mesh=mesh,
        scratch_types=[
            pltpu.VMEM((k,), I32),
            pltpu.VMEM((k, d), F32),
            pltpu.SemaphoreType.DMA,
        ],
    )
    def gk(table_hbm, idx_hbm, out_hbm, idx_v, rows_v, sem):
        wid = lax.axis_index("s") * nc + lax.axis_index("c")
        base = wid * per_w

        def body(j, carry):
            off = base + j * k
            pltpu.sync_copy(idx_hbm.at[pl.ds(off, k)], idx_v)
            pltpu.async_copy(table_hbm.at[idx_v], rows_v, sem).wait()
            pltpu.sync_copy(rows_v, out_hbm.at[pl.ds(off, k)])
            return carry

        lax.fori_loop(0, nblk, body, 0)

    return gk(table, idx)


def _sc_scatter_add(ext0, ext1, dst_idx, nd, k):
    e, dext = ext0.shape
    info = plsc.get_sparse_core_info()
    nc, ns = info.num_cores, info.num_subcores
    per_tile = e // ns
    nblk = per_tile // k
    rows_per_tile = ((nd + 8 * ns - 1) // (8 * ns)) * 8
    nd_pad = rows_per_tile * ns

    zeros = jnp.zeros((rows_per_tile, dext), F32)
    mesh = plsc.VectorSubcoreMesh(core_axis_name="c", subcore_axis_name="s")

    @functools.partial(
        pl.kernel,
        out_type=(
            jax.ShapeDtypeStruct((nd_pad, dext), F32),
            jax.ShapeDtypeStruct((nd_pad, dext), F32),
        ),
        mesh=mesh,
        scratch_types=[
            pltpu.VMEM((k,), I32),
            pltpu.VMEM((k, dext), F32),
            pltpu.SemaphoreType.DMA,
            pltpu.VMEM_SHARED((nd_pad, dext), F32),
        ],
    )
    def sk(e0_hbm, e1_hbm, dst_hbm, z_hbm, out0, out1, idx_v, rows_v, sem, acc):
        cid = lax.axis_index("c")
        sid = lax.axis_index("s")
        my_rows = pl.ds(sid * rows_per_tile, rows_per_tile)
        pltpu.sync_copy(z_hbm, acc.at[my_rows])
        plsc.subcore_barrier()

        base = sid * per_tile

        def body(ext_hbm):
            def step(j, carry):
                off = base + j * k
                pltpu.sync_copy(dst_hbm.at[pl.ds(off, k)], idx_v)
                pltpu.sync_copy(ext_hbm.at[pl.ds(off, k)], rows_v)
                pltpu.sync_copy(rows_v, acc.at[idx_v], add=True)
                return carry

            lax.fori_loop(0, nblk, step, 0)

        @pl.when(cid == 0)
        def _():
            body(e0_hbm)

        @pl.when(cid == 1)
        def _():
            body(e1_hbm)

        plsc.subcore_barrier()

        @pl.when(cid == 0)
        def _():
            pltpu.sync_copy(acc.at[my_rows], out0.at[my_rows])

        @pl.when(cid == 1)
        def _():
            pltpu.sync_copy(acc.at[my_rows], out1.at[my_rows])

    return sk(ext0, ext1, dst_idx, zeros)


def _tc_qn(src_data, wqn_t, bqp, nd, b):
    def body(q_ref, w_ref, b_ref, o_ref):
        o_ref[...] = (
            jnp.dot(q_ref[...], w_ref[...], preferred_element_type=F32) + b_ref[...]
        )

    return pl.pallas_call(
        body,
        grid=(nd // b,),
        in_specs=[
            pl.BlockSpec((b, 128), lambda i: (i, 0)),
            pl.BlockSpec((128, 128), lambda i: (0, 0)),
            pl.BlockSpec((1, 128), lambda i: (0, 0)),
        ],
        out_specs=pl.BlockSpec((b, 128), lambda i: (i, 0)),
        out_shape=jax.ShapeDtypeStruct((nd, 128), F32),
        compiler_params=pltpu.CompilerParams(
            dimension_semantics=("parallel",)
        ),
    )(src_data, wqn_t, bqp)


def _tc_edge_pass(src_data, qe, edge_feat, dt2, wk_parts, wv_parts, twp, tbp,
                  e, nd, b, dh, dext):
    wkn_t, wke_t, wkt_t, bk2 = wk_parts
    wvn_t, wve_t, wvt_t, bv2 = wv_parts
    row_off = nd // b
    de = edge_feat.shape[1]

    _C = (1.0, -19.739208221435547, 64.93939208984375, -85.4566879272461,
          60.24246597290039, -26.406761169433594, 7.8066086769104,
          -1.4609479904174805)

    def body(kv_ref, qe_ref, ef_ref, dt_ref,
             wkn, wke, wkt, bkr, wvn, wve, wvt, bvr, twr, tbr, o0_ref, o1_ref):
        u = dt_ref[...] * twr[...] + tbr[...]
        fr = u - jnp.floor(u + 0.5)
        t = fr * fr
        tf = _C[7]
        for c in (_C[6], _C[5], _C[4], _C[3], _C[2], _C[1], _C[0]):
            tf = tf * t + c
        kv = kv_ref[...]
        ef = ef_ref[...]
        k_mat = (
            jnp.dot(kv, wkn[...], preferred_element_type=F32)
            + jnp.dot(ef, wke[...], preferred_element_type=F32)
            + jnp.dot(tf, wkt[...], preferred_element_type=F32)
            + bkr[...]
        )
        v_mat = (
            jnp.dot(kv, wvn[...], preferred_element_type=F32)
            + jnp.dot(ef, wve[...], preferred_element_type=F32)
            + jnp.dot(tf, wvt[...], preferred_element_type=F32)
            + bvr[...]
        )
        s = qe_ref[...] * k_mat
        d0 = jnp.sum(s[:, :dh], axis=1, keepdims=True)
        d1 = jnp.sum(s[:, dh:], axis=1, keepdims=True)
        l0 = jnp.where(d0 >= 0, d0, 0.2 * d0)
        l1 = jnp.where(d1 >= 0, d1, 0.2 * d1)
        ex0 = jnp.exp(jnp.minimum(l0, 75.0))
        ex1 = jnp.exp(jnp.minimum(l1, 75.0))
        z = jnp.zeros((b, dext - dh - 1), F32)
        o0_ref[...] = jnp.concatenate(
            [v_mat[:, :dh] * jnp.broadcast_to(ex0, (b, dh)), ex0, z], axis=1
        )
        o1_ref[...] = jnp.concatenate(
            [v_mat[:, dh:] * jnp.broadcast_to(ex1, (b, dh)), ex1, z], axis=1
        )

    wspec = lambda shape: pl.BlockSpec(shape, lambda i: (0, 0))
    return pl.pallas_call(
        body,
        grid=(e // b,),
        in_specs=[
            pl.BlockSpec((b, 128), lambda i: (i + row_off, 0)),
            pl.BlockSpec((b, 128), lambda i: (i, 0)),
            pl.BlockSpec((b, de), lambda i: (i, 0)),
            pl.BlockSpec((b, 1), lambda i: (i, 0)),
            wspec((128, 128)), wspec((de, 128)), wspec((128, 128)), wspec((1, 128)),
            wspec((128, 128)), wspec((de, 128)), wspec((128, 128)), wspec((1, 128)),
            wspec((1, 128)), wspec((1, 128)),
        ],
        out_specs=(
            pl.BlockSpec((b, dext), lambda i: (i, 0)),
            pl.BlockSpec((b, dext), lambda i: (i, 0)),
        ),
        out_shape=(
            jax.ShapeDtypeStruct((e, dext), F32),
            jax.ShapeDtypeStruct((e, dext), F32),
        ),
        compiler_params=pltpu.CompilerParams(
            dimension_semantics=("parallel",)
        ),
    )(src_data, qe, edge_feat, dt2,
      wkn_t, wke_t, wkt_t, bk2, wvn_t, wve_t, wvt_t, bv2, twp, tbp)


def _tc_final(p0, p1, src_data, wod_t, wos_t, bo2, g2, b2, nd, b, dh, dext):
    def body(p0_ref, p1_ref, q_ref, wod, wos, bor, gr, br, o_ref):
        p0 = p0_ref[...]
        p1 = p1_ref[...]
        den0 = jnp.maximum(p0[:, dh : dh + 1], 1e-16)
        den1 = jnp.maximum(p1[:, dh : dh + 1], 1e-16)
        dst_h = jnp.concatenate(
            [p0[:, :dh] / jnp.broadcast_to(den0, (b, dh)),
             p1[:, :dh] / jnp.broadcast_to(den1, (b, dh))], axis=1
        )
        r = (
            jnp.dot(dst_h, wod[...], preferred_element_type=F32)
            + jnp.dot(q_ref[...], wos[...], preferred_element_type=F32)
            + bor[...]
        )
        r = jnp.maximum(r, 0.0)
        mu = jnp.mean(r, axis=1, keepdims=True)
        var = jnp.mean((r - mu) ** 2, axis=1, keepdims=True)
        o_ref[...] = (r - mu) / jnp.sqrt(var + 1e-5) * gr[...] + br[...]

    wspec = lambda shape: pl.BlockSpec(shape, lambda i: (0, 0))
    return pl.pallas_call(
        body,
        grid=(nd // b,),
        in_specs=[
            pl.BlockSpec((b, dext), lambda i: (i, 0)),
            pl.BlockSpec((b, dext), lambda i: (i, 0)),
            pl.BlockSpec((b, 128), lambda i: (i, 0)),
            wspec((128, 128)), wspec((128, 128)),
            wspec((1, 128)), wspec((1, 128)), wspec((1, 128)),
        ],
        out_specs=pl.BlockSpec((b, 128), lambda i: (i, 0)),
        out_shape=jax.ShapeDtypeStruct((nd, 128), F32),
        compiler_params=pltpu.CompilerParams(
            dimension_semantics=("parallel",)
        ),
    )(p0, p1, src_data, wod_t, wos_t, bo2, g2, b2)


def kernel(h, src_idx, edge_dt, edge_feat, edge_dst, num_dst, time_w, time_b,
           Wq, bq, Wk, bk, Wv, bv, Wo, bo, ln_g, ln_b):
    e = edge_dst.shape[0]
    n_src, dn = h.shape
    nd = n_src - e
    dt_dim = time_w.shape[0]
    dout = Wq.shape[0]
    dh = dout // 2
    de = edge_feat.shape[1]
    dext = 128
    b = 400
    k = 80
    nw = 32

    wqn_t = Wq[:, :dn].T
    bqp = (bq + jnp.cos(time_b) @ Wq[:, dn:].T).reshape(1, dout)
    wkn_t = Wk[:, :dn].T
    wke_t = Wk[:, dn : dn + de].T
    wkt_t = jnp.pad(Wk[:, dn + de :].T, ((0, 128 - dt_dim), (0, 0)))
    wvn_t = Wv[:, :dn].T
    wve_t = Wv[:, dn : dn + de].T
    wvt_t = jnp.pad(Wv[:, dn + de :].T, ((0, 128 - dt_dim), (0, 0)))
    inv2pi = 1.0 / (2.0 * jnp.pi)
    twp = (jnp.pad(time_w[:, 0], (0, 128 - dt_dim)) * inv2pi).reshape(1, 128)
    tbp = (jnp.pad(time_b, (0, 128 - dt_dim)) * inv2pi).reshape(1, 128)
    wod_t = Wo[:, :dout].T
    wos_t = Wo[:, dout:].T
    bo2 = bo.reshape(1, dout)
    g2 = ln_g.reshape(1, dout)
    b2 = ln_b.reshape(1, dout)
    bk2 = bk.reshape(1, dout)
    bv2 = bv.reshape(1, dout)

    chunk = nw * k
    n_pad = ((n_src + chunk - 1) // chunk) * chunk
    si = jnp.pad(src_idx.astype(I32), (0, n_pad - n_src))
    src_data = _sc_gather(h, si, k)

    qn = _tc_qn(src_data, wqn_t, bqp, nd, b)

    dst_i = edge_dst.astype(I32)
    qe = _sc_gather(qn, dst_i, k)

    dt2 = edge_dt.reshape(e, 1)
    ext0, ext1 = _tc_edge_pass(src_data, qe, edge_feat, dt2,
                               (wkn_t, wke_t, wkt_t, bk2), (wvn_t, wve_t, wvt_t, bv2),
                               twp, tbp, e, nd, b, dh, dext)

    p0, p1 = _sc_scatter_add(ext0, ext1, dst_i, nd, k)

    return _tc_final(p0, p1, src_data, wod_t, wos_t, bo2, g2, b2, nd, b, dh, dext)


# speedup vs baseline: 1.2515x; 1.0519x over previous
import functools

import jax
import jax.numpy as jnp
from jax import lax
from jax.experimental import pallas as pl
from jax.experimental.pallas import tpu as pltpu
from jax.experimental.pallas import tpu_sc as plsc

F32 = jnp.float32
I32 = jnp.int32


def _sc_gather(table, idx, k):
    n = idx.shape[0]
    d = table.shape[1]
    info = plsc.get_sparse_core_info()
    nc, ns = info.num_cores, info.num_subcores
    nw = nc * ns
    per_w = n // nw
    nblk = per_w // k
    assert nblk % 2 == 0

    mesh = plsc.VectorSubcoreMesh(core_axis_name="c", subcore_axis_name="s")

    @functools.partial(
        pl.kernel,
        out_type=jax.ShapeDtypeStruct((n, d), F32),
        mesh=mesh,
        scratch_types=[
            pltpu.VMEM((per_w,), I32),
            pltpu.VMEM((k, d), F32),
            pltpu.VMEM((k, d), F32),
            pltpu.SemaphoreType.DMA,
            pltpu.SemaphoreType.DMA,
            pltpu.SemaphoreType.DMA,
            pltpu.SemaphoreType.DMA,
        ],
    )
    def gk(table_hbm, idx_hbm, out_hbm, idx_all, rows0, rows1,
           semg0, semg1, semw0, semw1):
        wid = lax.axis_index("s") * nc + lax.axis_index("c")
        base = wid * per_w
        rows = (rows0, rows1)
        semg = (semg0, semg1)
        semw = (semw0, semw1)

        pltpu.sync_copy(idx_hbm.at[pl.ds(base, per_w)], idx_all)

        def gath(j, s):
            pltpu.async_copy(
                table_hbm.at[idx_all.at[pl.ds(j * k, k)]], rows[s], semg[s]
            )

        gath(0, 0)
        gath(1, 1)

        def body(j2, carry):
            for s in (0, 1):
                j = j2 * 2 + s
                pltpu.make_async_copy(rows[s], out_hbm.at[pl.ds(base + j * k, k)],
                                      semg[s]).wait()
                pltpu.async_copy(rows[s], out_hbm.at[pl.ds(base + j * k, k)],
                                 semw[s])
                pltpu.make_async_copy(rows[s], out_hbm.at[pl.ds(base + j * k, k)],
                                      semw[s]).wait()

                @pl.when(j + 2 < nblk)
                def _():
                    gath(j + 2, s)

            return carry

        lax.fori_loop(0, nblk // 2, body, 0)

    return gk(table, idx)


def _sc_scatter_add(ext0, ext1, dst_idx, nd, k):
    e, dext = ext0.shape
    info = plsc.get_sparse_core_info()
    nc, ns = info.num_cores, info.num_subcores
    per_tile = e // ns
    nblk = per_tile // k
    assert nblk % 2 == 0
    rows_per_tile = ((nd + 8 * ns - 1) // (8 * ns)) * 8
    nd_pad = rows_per_tile * ns

    zeros = jnp.zeros((rows_per_tile, dext), F32)
    mesh = plsc.VectorSubcoreMesh(core_axis_name="c", subcore_axis_name="s")

    @functools.partial(
        pl.kernel,
        out_type=(
            jax.ShapeDtypeStruct((nd_pad, dext), F32),
            jax.ShapeDtypeStruct((nd_pad, dext), F32),
        ),
        mesh=mesh,
        scratch_types=[
            pltpu.VMEM((k,), I32),
            pltpu.VMEM((k,), I32),
            pltpu.VMEM((k, dext), F32),
            pltpu.VMEM((k, dext), F32),
            pltpu.SemaphoreType.DMA,
            pltpu.SemaphoreType.DMA,
            pltpu.SemaphoreType.DMA,
            pltpu.SemaphoreType.DMA,
            pltpu.SemaphoreType.DMA,
            pltpu.SemaphoreType.DMA,
            pltpu.VMEM_SHARED((nd_pad, dext), F32),
        ],
    )
    def sk(e0_hbm, e1_hbm, dst_hbm, z_hbm, out0, out1, idx0, idx1, rows0, rows1,
           semi0, semi1, seml0, seml1, sems0, sems1, acc):
        cid = lax.axis_index("c")
        sid = lax.axis_index("s")
        my_rows = pl.ds(sid * rows_per_tile, rows_per_tile)
        idxv = (idx0, idx1)
        rows = (rows0, rows1)
        semi = (semi0, semi1)
        seml = (seml0, seml1)
        sems = (sems0, sems1)

        pltpu.sync_copy(z_hbm, acc.at[my_rows])
        plsc.subcore_barrier()

        base = sid * per_tile

        def body(ext_hbm):
            def load(j, s):
                pltpu.async_copy(dst_hbm.at[pl.ds(base + j * k, k)], idxv[s],
                                 semi[s])
                pltpu.async_copy(ext_hbm.at[pl.ds(base + j * k, k)], rows[s],
                                 seml[s])

            load(0, 0)
            load(1, 1)

            def step(j2, carry):
                for s in (0, 1):
                    j = j2 * 2 + s
                    pltpu.make_async_copy(dst_hbm.at[pl.ds(base, k)], idxv[s],
                                          semi[s]).wait()
                    pltpu.make_async_copy(ext_hbm.at[pl.ds(base, k)], rows[s],
                                          seml[s]).wait()
                    pltpu.async_copy(rows[s], acc.at[idxv[s]], sems[s],
                                     add=True)
                    pltpu.make_async_copy(ext_hbm.at[pl.ds(base, k)], rows[s],
                                          sems[s]).wait()

                    @pl.when(j + 2 < nblk)
                    def _():
                        load(j + 2, s)

                return carry

            lax.fori_loop(0, nblk // 2, step, 0)

        @pl.when(cid == 0)
        def _():
            body(e0_hbm)

        @pl.when(cid == 1)
        def _():
            body(e1_hbm)

        plsc.subcore_barrier()

        @pl.when(cid == 0)
        def _():
            pltpu.sync_copy(acc.at[my_rows], out0.at[my_rows])

        @pl.when(cid == 1)
        def _():
            pltpu.sync_copy(acc.at[my_rows], out1.at[my_rows])

    return sk(ext0, ext1, dst_idx, zeros)


def _tc_qn(src_data, wqn_t, bqp, nd, b):
    def body(q_ref, w_ref, b_ref, o_ref):
        o_ref[...] = (
            jnp.dot(q_ref[...], w_ref[...], preferred_element_type=F32) + b_ref[...]
        )

    return pl.pallas_call(
        body,
        grid=(nd // b,),
        in_specs=[
            pl.BlockSpec((b, 128), lambda i: (i, 0)),
            pl.BlockSpec((128, 128), lambda i: (0, 0)),
            pl.BlockSpec((1, 128), lambda i: (0, 0)),
        ],
        out_specs=pl.BlockSpec((b, 128), lambda i: (i, 0)),
        out_shape=jax.ShapeDtypeStruct((nd, 128), F32),
        compiler_params=pltpu.CompilerParams(
            dimension_semantics=("parallel",)
        ),
    )(src_data, wqn_t, bqp)


def _tc_edge_pass(src_data, qe, edge_feat, dt2, wk_parts, wv_parts, twp, tbp,
                  e, nd, b, dh, dext):
    wkn_t, wke_t, wkt_t, bk2 = wk_parts
    wvn_t, wve_t, wvt_t, bv2 = wv_parts
    row_off = nd // b
    de = edge_feat.shape[1]

    _C = (1.0, -19.739208221435547, 64.93939208984375, -85.4566879272461,
          60.24246597290039, -26.406761169433594, 7.8066086769104,
          -1.4609479904174805)

    def body(kv_ref, qe_ref, ef_ref, dt_ref,
             wkn, wke, wkt, bkr, wvn, wve, wvt, bvr, twr, tbr, o0_ref, o1_ref):
        u = dt_ref[...] * twr[...] + tbr[...]
        fr = u - jnp.floor(u + 0.5)
        t = fr * fr
        tf = _C[7]
        for c in (_C[6], _C[5], _C[4], _C[3], _C[2], _C[1], _C[0]):
            tf = tf * t + c
        kv = kv_ref[...]
        ef = ef_ref[...]
        k_mat = (
            jnp.dot(kv, wkn[...], preferred_element_type=F32)
            + jnp.dot(ef, wke[...], preferred_element_type=F32)
            + jnp.dot(tf, wkt[...], preferred_element_type=F32)
            + bkr[...]
        )
        v_mat = (
            jnp.dot(kv, wvn[...], preferred_element_type=F32)
            + jnp.dot(ef, wve[...], preferred_element_type=F32)
            + jnp.dot(tf, wvt[...], preferred_element_type=F32)
            + bvr[...]
        )
        s = qe_ref[...] * k_mat
        d0 = jnp.sum(s[:, :dh], axis=1, keepdims=True)
        d1 = jnp.sum(s[:, dh:], axis=1, keepdims=True)
        l0 = jnp.where(d0 >= 0, d0, 0.2 * d0)
        l1 = jnp.where(d1 >= 0, d1, 0.2 * d1)
        ex0 = jnp.exp(jnp.minimum(l0, 75.0))
        ex1 = jnp.exp(jnp.minimum(l1, 75.0))
        z = jnp.zeros((b, dext - dh - 1), F32)
        o0_ref[...] = jnp.concatenate(
            [v_mat[:, :dh] * jnp.broadcast_to(ex0, (b, dh)), ex0, z], axis=1
        )
        o1_ref[...] = jnp.concatenate(
            [v_mat[:, dh:] * jnp.broadcast_to(ex1, (b, dh)), ex1, z], axis=1
        )

    wspec = lambda shape: pl.BlockSpec(shape, lambda i: (0, 0))
    return pl.pallas_call(
        body,
        grid=(e // b,),
        in_specs=[
            pl.BlockSpec((b, 128), lambda i: (i + row_off, 0)),
            pl.BlockSpec((b, 128), lambda i: (i, 0)),
            pl.BlockSpec((b, de), lambda i: (i, 0)),
            pl.BlockSpec((b, 1), lambda i: (i, 0)),
            wspec((128, 128)), wspec((de, 128)), wspec((128, 128)), wspec((1, 128)),
            wspec((128, 128)), wspec((de, 128)), wspec((128, 128)), wspec((1, 128)),
            wspec((1, 128)), wspec((1, 128)),
        ],
        out_specs=(
            pl.BlockSpec((b, dext), lambda i: (i, 0)),
            pl.BlockSpec((b, dext), lambda i: (i, 0)),
        ),
        out_shape=(
            jax.ShapeDtypeStruct((e, dext), F32),
            jax.ShapeDtypeStruct((e, dext), F32),
        ),
        compiler_params=pltpu.CompilerParams(
            dimension_semantics=("parallel",)
        ),
    )(src_data, qe, edge_feat, dt2,
      wkn_t, wke_t, wkt_t, bk2, wvn_t, wve_t, wvt_t, bv2, twp, tbp)


def _tc_final(p0, p1, src_data, wod_t, wos_t, bo2, g2, b2, nd, b, dh, dext):
    def body(p0_ref, p1_ref, q_ref, wod, wos, bor, gr, br, o_ref):
        p0 = p0_ref[...]
        p1 = p1_ref[...]
        den0 = jnp.maximum(p0[:, dh : dh + 1], 1e-16)
        den1 = jnp.maximum(p1[:, dh : dh + 1], 1e-16)
        dst_h = jnp.concatenate(
            [p0[:, :dh] / jnp.broadcast_to(den0, (b, dh)),
             p1[:, :dh] / jnp.broadcast_to(den1, (b, dh))], axis=1
        )
        r = (
            jnp.dot(dst_h, wod[...], preferred_element_type=F32)
            + jnp.dot(q_ref[...], wos[...], preferred_element_type=F32)
            + bor[...]
        )
        r = jnp.maximum(r, 0.0)
        mu = jnp.mean(r, axis=1, keepdims=True)
        var = jnp.mean((r - mu) ** 2, axis=1, keepdims=True)
        o_ref[...] = (r - mu) / jnp.sqrt(var + 1e-5) * gr[...] + br[...]

    wspec = lambda shape: pl.BlockSpec(shape, lambda i: (0, 0))
    return pl.pallas_call(
        body,
        grid=(nd // b,),
        in_specs=[
            pl.BlockSpec((b, dext), lambda i: (i, 0)),
            pl.BlockSpec((b, dext), lambda i: (i, 0)),
            pl.BlockSpec((b, 128), lambda i: (i, 0)),
            wspec((128, 128)), wspec((128, 128)),
            wspec((1, 128)), wspec((1, 128)), wspec((1, 128)),
        ],
        out_specs=pl.BlockSpec((b, 128), lambda i: (i, 0)),
        out_shape=jax.ShapeDtypeStruct((nd, 128), F32),
        compiler_params=pltpu.CompilerParams(
            dimension_semantics=("parallel",)
        ),
    )(p0, p1, src_data, wod_t, wos_t, bo2, g2, b2)


def kernel(h, src_idx, edge_dt, edge_feat, edge_dst, num_dst, time_w, time_b,
           Wq, bq, Wk, bk, Wv, bv, Wo, bo, ln_g, ln_b):
    e = edge_dst.shape[0]
    n_src, dn = h.shape
    nd = n_src - e
    dt_dim = time_w.shape[0]
    dout = Wq.shape[0]
    dh = dout // 2
    de = edge_feat.shape[1]
    dext = 128
    b = 400
    k = 80
    nw = 32

    wqn_t = Wq[:, :dn].T
    bqp = (bq + jnp.cos(time_b) @ Wq[:, dn:].T).reshape(1, dout)
    wkn_t = Wk[:, :dn].T
    wke_t = Wk[:, dn : dn + de].T
    wkt_t = jnp.pad(Wk[:, dn + de :].T, ((0, 128 - dt_dim), (0, 0)))
    wvn_t = Wv[:, :dn].T
    wve_t = Wv[:, dn : dn + de].T
    wvt_t = jnp.pad(Wv[:, dn + de :].T, ((0, 128 - dt_dim), (0, 0)))
    inv2pi = 1.0 / (2.0 * jnp.pi)
    twp = (jnp.pad(time_w[:, 0], (0, 128 - dt_dim)) * inv2pi).reshape(1, 128)
    tbp = (jnp.pad(time_b, (0, 128 - dt_dim)) * inv2pi).reshape(1, 128)
    wod_t = Wo[:, :dout].T
    wos_t = Wo[:, dout:].T
    bo2 = bo.reshape(1, dout)
    g2 = ln_g.reshape(1, dout)
    b2 = ln_b.reshape(1, dout)
    bk2 = bk.reshape(1, dout)
    bv2 = bv.reshape(1, dout)

    k0 = 120
    chunk = nw * k0 * 2
    n_pad = ((n_src + chunk - 1) // chunk) * chunk
    si = jnp.pad(src_idx.astype(I32), (0, n_pad - n_src))
    src_data = _sc_gather(h, si, k0)

    qn = _tc_qn(src_data, wqn_t, bqp, nd, b)

    dst_i = edge_dst.astype(I32)
    k2 = 128
    chunk2 = nw * k2 * 2
    e_pad = ((e + chunk2 - 1) // chunk2) * chunk2
    qe = _sc_gather(qn, jnp.pad(dst_i, (0, e_pad - e)), k2)

    dt2 = edge_dt.reshape(e, 1)
    ext0, ext1 = _tc_edge_pass(src_data, qe, edge_feat, dt2,
                               (wkn_t, wke_t, wkt_t, bk2), (wvn_t, wve_t, wvt_t, bv2),
                               twp, tbp, e, nd, b, dh, dext)

    p0, p1 = _sc_scatter_add(ext0, ext1, dst_i, nd, k)

    return _tc_final(p0, p1, src_data, wod_t, wos_t, bo2, g2, b2, nd, b, dh, dext)


# speedup vs baseline: 1.7644x; 1.4098x over previous
import functools

import jax
import jax.numpy as jnp
from jax import lax
from jax.experimental import pallas as pl
from jax.experimental.pallas import tpu as pltpu
from jax.experimental.pallas import tpu_sc as plsc

F32 = jnp.float32
I32 = jnp.int32


def _sc_gather(table, idx, k):
    n = idx.shape[0]
    d = table.shape[1]
    info = plsc.get_sparse_core_info()
    nc, ns = info.num_cores, info.num_subcores
    nw = nc * ns
    per_w = n // nw
    nblk = per_w // k
    assert nblk % 2 == 0

    mesh = plsc.VectorSubcoreMesh(core_axis_name="c", subcore_axis_name="s")

    @functools.partial(
        pl.kernel,
        out_type=jax.ShapeDtypeStruct((n, d), F32),
        mesh=mesh,
        scratch_types=[
            pltpu.VMEM((per_w,), I32),
            pltpu.VMEM((k, d), F32),
            pltpu.VMEM((k, d), F32),
            pltpu.SemaphoreType.DMA,
            pltpu.SemaphoreType.DMA,
            pltpu.SemaphoreType.DMA,
            pltpu.SemaphoreType.DMA,
        ],
    )
    def gk(table_hbm, idx_hbm, out_hbm, idx_all, rows0, rows1,
           semg0, semg1, semw0, semw1):
        wid = lax.axis_index("s") * nc + lax.axis_index("c")
        base = wid * per_w
        rows = (rows0, rows1)
        semg = (semg0, semg1)
        semw = (semw0, semw1)

        pltpu.sync_copy(idx_hbm.at[pl.ds(base, per_w)], idx_all)

        def gath(j, s):
            pltpu.async_copy(
                table_hbm.at[idx_all.at[pl.ds(j * k, k)]], rows[s], semg[s]
            )

        gath(0, 0)
        gath(1, 1)

        def body(j2, carry):
            for s in (0, 1):
                j = j2 * 2 + s
                dst = out_hbm.at[pl.ds(base + j * k, k)]
                pltpu.make_async_copy(rows[s], dst, semg[s]).wait()
                pltpu.async_copy(rows[s], dst, semw[s])
                pltpu.make_async_copy(rows[s], dst, semw[s]).wait()

                @pl.when(j + 2 < nblk)
                def _():
                    gath(j + 2, s)

            return carry

        lax.fori_loop(0, nblk // 2, body, 0)

    return gk(table, idx)


def _sc_scatter_add(ext0, ext1, dst_idx, nd, k):
    e, dext = ext0.shape
    info = plsc.get_sparse_core_info()
    nc, ns = info.num_cores, info.num_subcores
    per_tile = e // ns
    nblk = per_tile // k
    assert nblk % 2 == 0
    rows_per_tile = ((nd + 8 * ns - 1) // (8 * ns)) * 8
    nd_pad = rows_per_tile * ns

    zeros = jnp.zeros((rows_per_tile, dext), F32)
    mesh = plsc.VectorSubcoreMesh(core_axis_name="c", subcore_axis_name="s")

    @functools.partial(
        pl.kernel,
        out_type=(
            jax.ShapeDtypeStruct((nd_pad, dext), F32),
            jax.ShapeDtypeStruct((nd_pad, dext), F32),
        ),
        mesh=mesh,
        scratch_types=[
            pltpu.VMEM((k,), I32),
            pltpu.VMEM((k,), I32),
            pltpu.VMEM((k, dext), F32),
            pltpu.VMEM((k, dext), F32),
            pltpu.SemaphoreType.DMA,
            pltpu.SemaphoreType.DMA,
            pltpu.SemaphoreType.DMA,
            pltpu.SemaphoreType.DMA,
            pltpu.SemaphoreType.DMA,
            pltpu.SemaphoreType.DMA,
            pltpu.VMEM_SHARED((nd_pad, dext), F32),
        ],
    )
    def sk(e0_hbm, e1_hbm, dst_hbm, z_hbm, out0, out1, idx0, idx1, rows0, rows1,
           semi0, semi1, seml0, seml1, sems0, sems1, acc):
        cid = lax.axis_index("c")
        sid = lax.axis_index("s")
        my_rows = pl.ds(sid * rows_per_tile, rows_per_tile)
        idxv = (idx0, idx1)
        rows = (rows0, rows1)
        semi = (semi0, semi1)
        seml = (seml0, seml1)
        sems = (sems0, sems1)

        pltpu.sync_copy(z_hbm, acc.at[my_rows])
        plsc.subcore_barrier()

        base = sid * per_tile

        def body(ext_hbm):
            def load(j, s):
                pltpu.async_copy(dst_hbm.at[pl.ds(base + j * k, k)], idxv[s],
                                 semi[s])
                pltpu.async_copy(ext_hbm.at[pl.ds(base + j * k, k)], rows[s],
                                 seml[s])

            load(0, 0)
            load(1, 1)

            def step(j2, carry):
                for s in (0, 1):
                    j = j2 * 2 + s
                    pltpu.make_async_copy(dst_hbm.at[pl.ds(base, k)], idxv[s],
                                          semi[s]).wait()
                    pltpu.make_async_copy(ext_hbm.at[pl.ds(base, k)], rows[s],
                                          seml[s]).wait()
                    pltpu.async_copy(rows[s], acc.at[idxv[s]], sems[s],
                                     add=True)
                    pltpu.make_async_copy(ext_hbm.at[pl.ds(base, k)], rows[s],
                                          sems[s]).wait()

                    @pl.when(j + 2 < nblk)
                    def _():
                        load(j + 2, s)

                return carry

            lax.fori_loop(0, nblk // 2, step, 0)

        @pl.when(cid == 0)
        def _():
            body(e0_hbm)

        @pl.when(cid == 1)
        def _():
            body(e1_hbm)

        plsc.subcore_barrier()

        @pl.when(cid == 0)
        def _():
            pltpu.sync_copy(acc.at[my_rows], out0.at[my_rows])

        @pl.when(cid == 1)
        def _():
            pltpu.sync_copy(acc.at[my_rows], out1.at[my_rows])

    return sk(ext0, ext1, dst_idx, zeros)


def _tc_edge_pass(src_data, qe_raw, edge_feat, dt2, wq_parts, wk_parts, wv_parts,
                  twp, tbp, e, nd, b, dh, dext):
    wqn_t, bqp = wq_parts
    wkn_t, wke_t, wkt_t, bk2 = wk_parts
    wvn_t, wve_t, wvt_t, bv2 = wv_parts
    row_off = nd // b
    de = edge_feat.shape[1]

    _C = (1.0, -19.739208221435547, 64.93939208984375, -85.4566879272461,
          60.24246597290039, -26.406761169433594, 7.8066086769104,
          -1.4609479904174805)

    def body(kv_ref, qe_ref, ef_ref, dt_ref,
             wqn, bqr, wkn, wke, wkt, bkr, wvn, wve, wvt, bvr, twr, tbr,
             o0_ref, o1_ref):
        u = dt_ref[...] * twr[...] + tbr[...]
        fr = u - jnp.floor(u + 0.5)
        t = fr * fr
        tf = _C[7]
        for c in (_C[6], _C[5], _C[4], _C[3], _C[2], _C[1], _C[0]):
            tf = tf * t + c
        kv = kv_ref[...]
        ef = ef_ref[...]
        q_mat = jnp.dot(qe_ref[...], wqn[...], preferred_element_type=F32) + bqr[...]
        k_mat = (
            jnp.dot(kv, wkn[...], preferred_element_type=F32)
            + jnp.dot(ef, wke[...], preferred_element_type=F32)
            + jnp.dot(tf, wkt[...], preferred_element_type=F32)
            + bkr[...]
        )
        v_mat = (
            jnp.dot(kv, wvn[...], preferred_element_type=F32)
            + jnp.dot(ef, wve[...], preferred_element_type=F32)
            + jnp.dot(tf, wvt[...], preferred_element_type=F32)
            + bvr[...]
        )
        s = q_mat * k_mat
        ones_h = jnp.ones((dh, 1), F32)
        d0 = jnp.dot(s[:, :dh], ones_h, preferred_element_type=F32)
        d1 = jnp.dot(s[:, dh:], ones_h, preferred_element_type=F32)
        l0 = jnp.where(d0 >= 0, d0, 0.2 * d0)
        l1 = jnp.where(d1 >= 0, d1, 0.2 * d1)
        ex0 = jnp.exp(jnp.minimum(l0, 75.0))
        ex1 = jnp.exp(jnp.minimum(l1, 75.0))
        z = jnp.zeros((b, dext - dh - 1), F32)
        o0_ref[...] = jnp.concatenate(
            [v_mat[:, :dh] * jnp.broadcast_to(ex0, (b, dh)), ex0, z], axis=1
        )
        o1_ref[...] = jnp.concatenate(
            [v_mat[:, dh:] * jnp.broadcast_to(ex1, (b, dh)), ex1, z], axis=1
        )

    wspec = lambda shape: pl.BlockSpec(shape, lambda i: (0, 0))
    return pl.pallas_call(
        body,
        grid=(e // b,),
        in_specs=[
            pl.BlockSpec((b, 128), lambda i: (i + row_off, 0)),
            pl.BlockSpec((b, 128), lambda i: (i, 0)),
            pl.BlockSpec((b, de), lambda i: (i, 0)),
            pl.BlockSpec((b, 1), lambda i: (i, 0)),
            wspec((128, 128)), wspec((1, 128)),
            wspec((128, 128)), wspec((de, 128)), wspec((128, 128)), wspec((1, 128)),
            wspec((128, 128)), wspec((de, 128)), wspec((128, 128)), wspec((1, 128)),
            wspec((1, 128)), wspec((1, 128)),
        ],
        out_specs=(
            pl.BlockSpec((b, dext), lambda i: (i, 0)),
            pl.BlockSpec((b, dext), lambda i: (i, 0)),
        ),
        out_shape=(
            jax.ShapeDtypeStruct((e, dext), F32),
            jax.ShapeDtypeStruct((e, dext), F32),
        ),
        compiler_params=pltpu.CompilerParams(
            dimension_semantics=("parallel",)
        ),
    )(src_data, qe_raw, edge_feat, dt2, wqn_t, bqp,
      wkn_t, wke_t, wkt_t, bk2, wvn_t, wve_t, wvt_t, bv2, twp, tbp)


def _tc_final(p0, p1, src_data, wod_t, wos_t, bo2, g2, b2, nd, b, dh, dext):
    def body(p0_ref, p1_ref, q_ref, wod, wos, bor, gr, br, o_ref):
        p0 = p0_ref[...]
        p1 = p1_ref[...]
        den0 = jnp.maximum(p0[:, dh : dh + 1], 1e-16)
        den1 = jnp.maximum(p1[:, dh : dh + 1], 1e-16)
        dst_h = jnp.concatenate(
            [p0[:, :dh] / jnp.broadcast_to(den0, (b, dh)),
             p1[:, :dh] / jnp.broadcast_to(den1, (b, dh))], axis=1
        )
        r = (
            jnp.dot(dst_h, wod[...], preferred_element_type=F32)
            + jnp.dot(q_ref[...], wos[...], preferred_element_type=F32)
            + bor[...]
        )
        r = jnp.maximum(r, 0.0)
        mu = jnp.mean(r, axis=1, keepdims=True)
        var = jnp.mean((r - mu) ** 2, axis=1, keepdims=True)
        o_ref[...] = (r - mu) / jnp.sqrt(var + 1e-5) * gr[...] + br[...]

    wspec = lambda shape: pl.BlockSpec(shape, lambda i: (0, 0))
    return pl.pallas_call(
        body,
        grid=(nd // b,),
        in_specs=[
            pl.BlockSpec((b, dext), lambda i: (i, 0)),
            pl.BlockSpec((b, dext), lambda i: (i, 0)),
            pl.BlockSpec((b, 128), lambda i: (i, 0)),
            wspec((128, 128)), wspec((128, 128)),
            wspec((1, 128)), wspec((1, 128)), wspec((1, 128)),
        ],
        out_specs=pl.BlockSpec((b, 128), lambda i: (i, 0)),
        out_shape=jax.ShapeDtypeStruct((nd, 128), F32),
        compiler_params=pltpu.CompilerParams(
            dimension_semantics=("parallel",)
        ),
    )(p0, p1, src_data, wod_t, wos_t, bo2, g2, b2)


def kernel(h, src_idx, edge_dt, edge_feat, edge_dst, num_dst, time_w, time_b,
           Wq, bq, Wk, bk, Wv, bv, Wo, bo, ln_g, ln_b):
    e = edge_dst.shape[0]
    n_src, dn = h.shape
    nd = n_src - e
    dt_dim = time_w.shape[0]
    dout = Wq.shape[0]
    dh = dout // 2
    de = edge_feat.shape[1]
    dext = 128
    b = 400
    k = 80
    nw = 32

    wqn_t = Wq[:, :dn].T
    bqp = (bq + jnp.cos(time_b) @ Wq[:, dn:].T).reshape(1, dout)
    wkn_t = Wk[:, :dn].T
    wke_t = Wk[:, dn : dn + de].T
    wkt_t = jnp.pad(Wk[:, dn + de :].T, ((0, 128 - dt_dim), (0, 0)))
    wvn_t = Wv[:, :dn].T
    wve_t = Wv[:, dn : dn + de].T
    wvt_t = jnp.pad(Wv[:, dn + de :].T, ((0, 128 - dt_dim), (0, 0)))
    inv2pi = 1.0 / (2.0 * jnp.pi)
    twp = (jnp.pad(time_w[:, 0], (0, 128 - dt_dim)) * inv2pi).reshape(1, 128)
    tbp = (jnp.pad(time_b, (0, 128 - dt_dim)) * inv2pi).reshape(1, 128)
    wod_t = Wo[:, :dout].T
    wos_t = Wo[:, dout:].T
    bo2 = bo.reshape(1, dout)
    g2 = ln_g.reshape(1, dout)
    b2 = ln_b.reshape(1, dout)
    bk2 = bk.reshape(1, dout)
    bv2 = bv.reshape(1, dout)

    k0 = 120
    chunk = nw * k0 * 2
    n_pad = ((n_src + chunk - 1) // chunk) * chunk
    si = jnp.pad(src_idx.astype(I32), (0, n_pad - n_src))
    src_data = _sc_gather(h, si, k0)

    dst_i = edge_dst.astype(I32)
    k2 = 128
    chunk2 = nw * k2 * 2
    e_pad = ((e + chunk2 - 1) // chunk2) * chunk2
    dst_pad = jnp.pad(dst_i, (0, e_pad - e))
    qe_raw = _sc_gather(src_data, dst_pad, k2)

    b3 = 2000
    dt2 = edge_dt.reshape(e, 1)
    ext0, ext1 = _tc_edge_pass(src_data, qe_raw, edge_feat, dt2,
                               (wqn_t, bqp),
                               (wkn_t, wke_t, wkt_t, bk2), (wvn_t, wve_t, wvt_t, bv2),
                               twp, tbp, e, nd, b3, dh, dext)

    p0, p1 = _sc_scatter_add(ext0, ext1, dst_i, nd, k)

    return _tc_final(p0, p1, src_data, wod_t, wos_t, bo2, g2, b2, nd, b, dh, dext)


# speedup vs baseline: 2.3269x; 1.3188x over previous
import functools

import jax
import jax.numpy as jnp
from jax import lax
from jax.experimental import pallas as pl
from jax.experimental.pallas import tpu as pltpu
from jax.experimental.pallas import tpu_sc as plsc

F32 = jnp.float32
I32 = jnp.int32


def _sc_gather(table, idx, k):
    n = idx.shape[0]
    d = table.shape[1]
    info = plsc.get_sparse_core_info()
    nc, ns = info.num_cores, info.num_subcores
    nw = nc * ns
    per_w = n // nw
    nblk = per_w // k
    assert nblk % 2 == 0

    mesh = plsc.VectorSubcoreMesh(core_axis_name="c", subcore_axis_name="s")

    @functools.partial(
        pl.kernel,
        out_type=jax.ShapeDtypeStruct((n, d), F32),
        mesh=mesh,
        scratch_types=[
            pltpu.VMEM((per_w,), I32),
            pltpu.VMEM((k, d), F32),
            pltpu.VMEM((k, d), F32),
            pltpu.SemaphoreType.DMA,
            pltpu.SemaphoreType.DMA,
            pltpu.SemaphoreType.DMA,
            pltpu.SemaphoreType.DMA,
        ],
    )
    def gk(table_hbm, idx_hbm, out_hbm, idx_all, rows0, rows1,
           semg0, semg1, semw0, semw1):
        wid = lax.axis_index("s") * nc + lax.axis_index("c")
        base = wid * per_w
        rows = (rows0, rows1)
        semg = (semg0, semg1)
        semw = (semw0, semw1)

        pltpu.sync_copy(idx_hbm.at[pl.ds(base, per_w)], idx_all)

        def gath(j, s):
            pltpu.async_copy(
                table_hbm.at[idx_all.at[pl.ds(j * k, k)]], rows[s], semg[s]
            )

        gath(0, 0)
        gath(1, 1)

        def body(j2, carry):
            for s in (0, 1):
                j = j2 * 2 + s
                dst = out_hbm.at[pl.ds(base + j * k, k)]
                pltpu.make_async_copy(rows[s], dst, semg[s]).wait()
                pltpu.async_copy(rows[s], dst, semw[s])
                pltpu.make_async_copy(rows[s], dst, semw[s]).wait()

                @pl.when(j + 2 < nblk)
                def _():
                    gath(j + 2, s)

            return carry

        lax.fori_loop(0, nblk // 2, body, 0)

    return gk(table, idx)


def _sc_gather_small(table, idx, k, t_rows):
    n = idx.shape[0]
    d = table.shape[1]
    info = plsc.get_sparse_core_info()
    nc, ns = info.num_cores, info.num_subcores
    nw = nc * ns
    per_w = n // nw
    nblk = per_w // k
    assert nblk % 2 == 0 and t_rows % (8 * ns) == 0
    stage = t_rows // ns

    mesh = plsc.VectorSubcoreMesh(core_axis_name="c", subcore_axis_name="s")

    @functools.partial(
        pl.kernel,
        out_type=jax.ShapeDtypeStruct((n, d), F32),
        mesh=mesh,
        scratch_types=[
            pltpu.VMEM((per_w,), I32),
            pltpu.VMEM((k, d), F32),
            pltpu.VMEM((k, d), F32),
            pltpu.SemaphoreType.DMA,
            pltpu.SemaphoreType.DMA,
            pltpu.SemaphoreType.DMA,
            pltpu.SemaphoreType.DMA,
            pltpu.VMEM_SHARED((t_rows, d), F32),
        ],
    )
    def gk(table_hbm, idx_hbm, out_hbm, idx_all, rows0, rows1,
           semg0, semg1, semw0, semw1, spm):
        sid = lax.axis_index("s")
        wid = sid * nc + lax.axis_index("c")
        base = wid * per_w
        rows = (rows0, rows1)
        semg = (semg0, semg1)
        semw = (semw0, semw1)

        my_stage = pl.ds(sid * stage, stage)
        pltpu.sync_copy(table_hbm.at[my_stage], spm.at[my_stage])
        pltpu.sync_copy(idx_hbm.at[pl.ds(base, per_w)], idx_all)
        plsc.subcore_barrier()

        def gath(j, s):
            pltpu.async_copy(
                spm.at[idx_all.at[pl.ds(j * k, k)]], rows[s], semg[s]
            )

        gath(0, 0)
        gath(1, 1)

        def body(j2, carry):
            for s in (0, 1):
                j = j2 * 2 + s
                dst = out_hbm.at[pl.ds(base + j * k, k)]
                pltpu.make_async_copy(rows[s], dst, semg[s]).wait()
                pltpu.async_copy(rows[s], dst, semw[s])
                pltpu.make_async_copy(rows[s], dst, semw[s]).wait()

                @pl.when(j + 2 < nblk)
                def _():
                    gath(j + 2, s)

            return carry

        lax.fori_loop(0, nblk // 2, body, 0)

    return gk(table, idx)


def _sc_scatter_add(ext0, ext1, dst_idx, nd, k):
    e, dext = ext0.shape
    info = plsc.get_sparse_core_info()
    nc, ns = info.num_cores, info.num_subcores
    per_tile = e // ns
    nblk = per_tile // k
    assert nblk % 2 == 0
    rows_per_tile = ((nd + 8 * ns - 1) // (8 * ns)) * 8
    nd_pad = rows_per_tile * ns

    zeros = jnp.zeros((rows_per_tile, dext), F32)
    mesh = plsc.VectorSubcoreMesh(core_axis_name="c", subcore_axis_name="s")

    @functools.partial(
        pl.kernel,
        out_type=(
            jax.ShapeDtypeStruct((nd_pad, dext), F32),
            jax.ShapeDtypeStruct((nd_pad, dext), F32),
        ),
        mesh=mesh,
        scratch_types=[
            pltpu.VMEM((k,), I32),
            pltpu.VMEM((k,), I32),
            pltpu.VMEM((k, dext), F32),
            pltpu.VMEM((k, dext), F32),
            pltpu.SemaphoreType.DMA,
            pltpu.SemaphoreType.DMA,
            pltpu.SemaphoreType.DMA,
            pltpu.SemaphoreType.DMA,
            pltpu.SemaphoreType.DMA,
            pltpu.SemaphoreType.DMA,
            pltpu.VMEM_SHARED((nd_pad, dext), F32),
        ],
    )
    def sk(e0_hbm, e1_hbm, dst_hbm, z_hbm, out0, out1, idx0, idx1, rows0, rows1,
           semi0, semi1, seml0, seml1, sems0, sems1, acc):
        cid = lax.axis_index("c")
        sid = lax.axis_index("s")
        my_rows = pl.ds(sid * rows_per_tile, rows_per_tile)
        idxv = (idx0, idx1)
        rows = (rows0, rows1)
        semi = (semi0, semi1)
        seml = (seml0, seml1)
        sems = (sems0, sems1)

        pltpu.sync_copy(z_hbm, acc.at[my_rows])
        plsc.subcore_barrier()

        base = sid * per_tile

        def body(ext_hbm):
            def load(j, s):
                pltpu.async_copy(dst_hbm.at[pl.ds(base + j * k, k)], idxv[s],
                                 semi[s])
                pltpu.async_copy(ext_hbm.at[pl.ds(base + j * k, k)], rows[s],
                                 seml[s])

            load(0, 0)
            load(1, 1)

            def step(j2, carry):
                for s in (0, 1):
                    j = j2 * 2 + s
                    pltpu.make_async_copy(dst_hbm.at[pl.ds(base, k)], idxv[s],
                                          semi[s]).wait()
                    pltpu.make_async_copy(ext_hbm.at[pl.ds(base, k)], rows[s],
                                          seml[s]).wait()
                    pltpu.async_copy(rows[s], acc.at[idxv[s]], sems[s],
                                     add=True)
                    pltpu.make_async_copy(ext_hbm.at[pl.ds(base, k)], rows[s],
                                          sems[s]).wait()

                    @pl.when(j + 2 < nblk)
                    def _():
                        load(j + 2, s)

                return carry

            lax.fori_loop(0, nblk // 2, step, 0)

        @pl.when(cid == 0)
        def _():
            body(e0_hbm)

        @pl.when(cid == 1)
        def _():
            body(e1_hbm)

        plsc.subcore_barrier()

        @pl.when(cid == 0)
        def _():
            pltpu.sync_copy(acc.at[my_rows], out0.at[my_rows])

        @pl.when(cid == 1)
        def _():
            pltpu.sync_copy(acc.at[my_rows], out1.at[my_rows])

    return sk(ext0, ext1, dst_idx, zeros)


def _tc_edge_pass(src_data, qe_raw, edge_feat, dt2, wq_parts, wk_parts, wv_parts,
                  twp, tbp, e, nd, b, dh, dext):
    wqn_t, bqp = wq_parts
    wkn_t, wke_t, wkt_t, bk2 = wk_parts
    wvn_t, wve_t, wvt_t, bv2 = wv_parts
    row_off = nd // b
    de = edge_feat.shape[1]

    _C = (1.0, -19.739208221435547, 64.93939208984375, -85.4566879272461,
          60.24246597290039, -26.406761169433594, 7.8066086769104,
          -1.4609479904174805)

    def body(kv_ref, qe_ref, ef_ref, dt_ref,
             wqn, bqr, wkn, wke, wkt, bkr, wvn, wve, wvt, bvr, twr, tbr,
             o0_ref, o1_ref):
        u = dt_ref[...] * twr[...] + tbr[...]
        fr = u - jnp.floor(u + 0.5)
        t = fr * fr
        tf = _C[7]
        for c in (_C[6], _C[5], _C[4], _C[3], _C[2], _C[1], _C[0]):
            tf = tf * t + c
        kv = kv_ref[...]
        ef = ef_ref[...]
        q_mat = jnp.dot(qe_ref[...], wqn[...], preferred_element_type=F32) + bqr[...]
        k_mat = (
            jnp.dot(kv, wkn[...], preferred_element_type=F32)
            + jnp.dot(ef, wke[...], preferred_element_type=F32)
            + jnp.dot(tf, wkt[...], preferred_element_type=F32)
            + bkr[...]
        )
        v_mat = (
            jnp.dot(kv, wvn[...], preferred_element_type=F32)
            + jnp.dot(ef, wve[...], preferred_element_type=F32)
            + jnp.dot(tf, wvt[...], preferred_element_type=F32)
            + bvr[...]
        )
        s = q_mat * k_mat
        ones_h = jnp.ones((dh, 1), F32)
        d0 = jnp.dot(s[:, :dh], ones_h, preferred_element_type=F32)
        d1 = jnp.dot(s[:, dh:], ones_h, preferred_element_type=F32)
        l0 = jnp.where(d0 >= 0, d0, 0.2 * d0)
        l1 = jnp.where(d1 >= 0, d1, 0.2 * d1)
        ex0 = jnp.exp(jnp.minimum(l0, 75.0))
        ex1 = jnp.exp(jnp.minimum(l1, 75.0))
        z = jnp.zeros((b, dext - dh - 1), F32)
        o0_ref[...] = jnp.concatenate(
            [v_mat[:, :dh] * jnp.broadcast_to(ex0, (b, dh)), ex0, z], axis=1
        )
        o1_ref[...] = jnp.concatenate(
            [v_mat[:, dh:] * jnp.broadcast_to(ex1, (b, dh)), ex1, z], axis=1
        )

    wspec = lambda shape: pl.BlockSpec(shape, lambda i: (0, 0))
    return pl.pallas_call(
        body,
        grid=(e // b,),
        in_specs=[
            pl.BlockSpec((b, 128), lambda i: (i + row_off, 0)),
            pl.BlockSpec((b, 128), lambda i: (i, 0)),
            pl.BlockSpec((b, de), lambda i: (i, 0)),
            pl.BlockSpec((b, 1), lambda i: (i, 0)),
            wspec((128, 128)), wspec((1, 128)),
            wspec((128, 128)), wspec((de, 128)), wspec((128, 128)), wspec((1, 128)),
            wspec((128, 128)), wspec((de, 128)), wspec((128, 128)), wspec((1, 128)),
            wspec((1, 128)), wspec((1, 128)),
        ],
        out_specs=(
            pl.BlockSpec((b, dext), lambda i: (i, 0)),
            pl.BlockSpec((b, dext), lambda i: (i, 0)),
        ),
        out_shape=(
            jax.ShapeDtypeStruct((e, dext), F32),
            jax.ShapeDtypeStruct((e, dext), F32),
        ),
        compiler_params=pltpu.CompilerParams(
            dimension_semantics=("parallel",)
        ),
    )(src_data, qe_raw, edge_feat, dt2, wqn_t, bqp,
      wkn_t, wke_t, wkt_t, bk2, wvn_t, wve_t, wvt_t, bv2, twp, tbp)


def _tc_final(p0, p1, src_data, wod_t, wos_t, bo2, g2, b2, nd, b, dh, dext):
    def body(p0_ref, p1_ref, q_ref, wod, wos, bor, gr, br, o_ref):
        p0 = p0_ref[...]
        p1 = p1_ref[...]
        den0 = jnp.maximum(p0[:, dh : dh + 1], 1e-16)
        den1 = jnp.maximum(p1[:, dh : dh + 1], 1e-16)
        dst_h = jnp.concatenate(
            [p0[:, :dh] / jnp.broadcast_to(den0, (b, dh)),
             p1[:, :dh] / jnp.broadcast_to(den1, (b, dh))], axis=1
        )
        r = (
            jnp.dot(dst_h, wod[...], preferred_element_type=F32)
            + jnp.dot(q_ref[...], wos[...], preferred_element_type=F32)
            + bor[...]
        )
        r = jnp.maximum(r, 0.0)
        mu = jnp.mean(r, axis=1, keepdims=True)
        var = jnp.mean((r - mu) ** 2, axis=1, keepdims=True)
        o_ref[...] = (r - mu) / jnp.sqrt(var + 1e-5) * gr[...] + br[...]

    wspec = lambda shape: pl.BlockSpec(shape, lambda i: (0, 0))
    return pl.pallas_call(
        body,
        grid=(nd // b,),
        in_specs=[
            pl.BlockSpec((b, dext), lambda i: (i, 0)),
            pl.BlockSpec((b, dext), lambda i: (i, 0)),
            pl.BlockSpec((b, 128), lambda i: (i, 0)),
            wspec((128, 128)), wspec((128, 128)),
            wspec((1, 128)), wspec((1, 128)), wspec((1, 128)),
        ],
        out_specs=pl.BlockSpec((b, 128), lambda i: (i, 0)),
        out_shape=jax.ShapeDtypeStruct((nd, 128), F32),
        compiler_params=pltpu.CompilerParams(
            dimension_semantics=("parallel",)
        ),
    )(p0, p1, src_data, wod_t, wos_t, bo2, g2, b2)


def kernel(h, src_idx, edge_dt, edge_feat, edge_dst, num_dst, time_w, time_b,
           Wq, bq, Wk, bk, Wv, bv, Wo, bo, ln_g, ln_b):
    e = edge_dst.shape[0]
    n_src, dn = h.shape
    nd = n_src - e
    dt_dim = time_w.shape[0]
    dout = Wq.shape[0]
    dh = dout // 2
    de = edge_feat.shape[1]
    dext = 128
    b = 400
    k = 80
    nw = 32

    wqn_t = Wq[:, :dn].T
    bqp = (bq + jnp.cos(time_b) @ Wq[:, dn:].T).reshape(1, dout)
    wkn_t = Wk[:, :dn].T
    wke_t = Wk[:, dn : dn + de].T
    wkt_t = jnp.pad(Wk[:, dn + de :].T, ((0, 128 - dt_dim), (0, 0)))
    wvn_t = Wv[:, :dn].T
    wve_t = Wv[:, dn : dn + de].T
    wvt_t = jnp.pad(Wv[:, dn + de :].T, ((0, 128 - dt_dim), (0, 0)))
    inv2pi = 1.0 / (2.0 * jnp.pi)
    twp = (jnp.pad(time_w[:, 0], (0, 128 - dt_dim)) * inv2pi).reshape(1, 128)
    tbp = (jnp.pad(time_b, (0, 128 - dt_dim)) * inv2pi).reshape(1, 128)
    wod_t = Wo[:, :dout].T
    wos_t = Wo[:, dout:].T
    bo2 = bo.reshape(1, dout)
    g2 = ln_g.reshape(1, dout)
    b2 = ln_b.reshape(1, dout)
    bk2 = bk.reshape(1, dout)
    bv2 = bv.reshape(1, dout)

    k0 = 120
    chunk = nw * k0 * 2
    n_pad = ((n_src + chunk - 1) // chunk) * chunk
    si = jnp.pad(src_idx.astype(I32), (0, n_pad - n_src))
    src_data = _sc_gather(h, si, k0)

    dst_i = edge_dst.astype(I32)
    k2 = 128
    chunk2 = nw * k2 * 2
    e_pad = ((e + chunk2 - 1) // chunk2) * chunk2
    dst_pad = jnp.pad(dst_i, (0, e_pad - e))
    qe_raw = _sc_gather_small(src_data, dst_pad, k2, 10240)

    b3 = 2000
    dt2 = edge_dt.reshape(e, 1)
    ext0, ext1 = _tc_edge_pass(src_data, qe_raw, edge_feat, dt2,
                               (wqn_t, bqp),
                               (wkn_t, wke_t, wkt_t, bk2), (wvn_t, wve_t, wvt_t, bv2),
                               twp, tbp, e, nd, b3, dh, dext)

    p0, p1 = _sc_scatter_add(ext0, ext1, dst_i, nd, k)

    return _tc_final(p0, p1, src_data, wod_t, wos_t, bo2, g2, b2, nd, b, dh, dext)


# speedup vs baseline: 2.3356x; 1.0037x over previous
import functools

import jax
import jax.numpy as jnp
from jax import lax
from jax.experimental import pallas as pl
from jax.experimental.pallas import tpu as pltpu
from jax.experimental.pallas import tpu_sc as plsc

F32 = jnp.float32
I32 = jnp.int32


def _sc_gather(table, idx, k):
    n = idx.shape[0]
    d = table.shape[1]
    info = plsc.get_sparse_core_info()
    nc, ns = info.num_cores, info.num_subcores
    nw = nc * ns
    per_w = n // nw
    nblk = per_w // k
    assert nblk % 2 == 0

    mesh = plsc.VectorSubcoreMesh(core_axis_name="c", subcore_axis_name="s")

    @functools.partial(
        pl.kernel,
        out_type=jax.ShapeDtypeStruct((n, d), F32),
        mesh=mesh,
        scratch_types=[
            pltpu.VMEM((per_w,), I32),
            pltpu.VMEM((k, d), F32),
            pltpu.VMEM((k, d), F32),
            pltpu.SemaphoreType.DMA,
            pltpu.SemaphoreType.DMA,
            pltpu.SemaphoreType.DMA,
            pltpu.SemaphoreType.DMA,
        ],
    )
    def gk(table_hbm, idx_hbm, out_hbm, idx_all, rows0, rows1,
           semg0, semg1, semw0, semw1):
        wid = lax.axis_index("s") * nc + lax.axis_index("c")
        base = wid * per_w
        rows = (rows0, rows1)
        semg = (semg0, semg1)
        semw = (semw0, semw1)

        pltpu.sync_copy(idx_hbm.at[pl.ds(base, per_w)], idx_all)

        def gath(j, s):
            pltpu.async_copy(
                table_hbm.at[idx_all.at[pl.ds(j * k, k)]], rows[s], semg[s]
            )

        gath(0, 0)
        gath(1, 1)

        def body(j2, carry):
            for s in (0, 1):
                j = j2 * 2 + s
                dst = out_hbm.at[pl.ds(base + j * k, k)]
                pltpu.make_async_copy(rows[s], dst, semg[s]).wait()
                pltpu.async_copy(rows[s], dst, semw[s])
                pltpu.make_async_copy(rows[s], dst, semw[s]).wait()

                @pl.when(j + 2 < nblk)
                def _():
                    gath(j + 2, s)

            return carry

        lax.fori_loop(0, nblk // 2, body, 0)

    return gk(table, idx)


def _sc_gather_small(table, idx, k, t_rows):
    n = idx.shape[0]
    d = table.shape[1]
    info = plsc.get_sparse_core_info()
    nc, ns = info.num_cores, info.num_subcores
    nw = nc * ns
    per_w = n // nw
    nblk = per_w // k
    assert nblk % 2 == 0 and t_rows % (8 * ns) == 0
    stage = t_rows // ns

    mesh = plsc.VectorSubcoreMesh(core_axis_name="c", subcore_axis_name="s")

    @functools.partial(
        pl.kernel,
        out_type=jax.ShapeDtypeStruct((n, d), F32),
        mesh=mesh,
        scratch_types=[
            pltpu.VMEM((per_w,), I32),
            pltpu.VMEM((k, d), F32),
            pltpu.VMEM((k, d), F32),
            pltpu.SemaphoreType.DMA,
            pltpu.SemaphoreType.DMA,
            pltpu.SemaphoreType.DMA,
            pltpu.SemaphoreType.DMA,
            pltpu.VMEM_SHARED((t_rows, d), F32),
        ],
    )
    def gk(table_hbm, idx_hbm, out_hbm, idx_all, rows0, rows1,
           semg0, semg1, semw0, semw1, spm):
        sid = lax.axis_index("s")
        wid = sid * nc + lax.axis_index("c")
        base = wid * per_w
        rows = (rows0, rows1)
        semg = (semg0, semg1)
        semw = (semw0, semw1)

        my_stage = pl.ds(sid * stage, stage)
        pltpu.sync_copy(table_hbm.at[my_stage], spm.at[my_stage])
        pltpu.sync_copy(idx_hbm.at[pl.ds(base, per_w)], idx_all)
        plsc.subcore_barrier()

        def gath(j, s):
            pltpu.async_copy(
                spm.at[idx_all.at[pl.ds(j * k, k)]], rows[s], semg[s]
            )

        gath(0, 0)
        gath(1, 1)

        def body(j2, carry):
            for s in (0, 1):
                j = j2 * 2 + s
                dst = out_hbm.at[pl.ds(base + j * k, k)]
                pltpu.make_async_copy(rows[s], dst, semg[s]).wait()
                pltpu.async_copy(rows[s], dst, semw[s])
                pltpu.make_async_copy(rows[s], dst, semw[s]).wait()

                @pl.when(j + 2 < nblk)
                def _():
                    gath(j + 2, s)

            return carry

        lax.fori_loop(0, nblk // 2, body, 0)

    return gk(table, idx)


def _sc_scatter_add(ext0, ext1, dst_idx, nd, k):
    e, dext = ext0.shape
    info = plsc.get_sparse_core_info()
    nc, ns = info.num_cores, info.num_subcores
    per_tile = e // ns
    nblk = per_tile // k
    assert nblk % 2 == 0
    rows_per_tile = ((nd + 8 * ns - 1) // (8 * ns)) * 8
    nd_pad = rows_per_tile * ns

    zeros = jnp.zeros((rows_per_tile, dext), F32)
    mesh = plsc.VectorSubcoreMesh(core_axis_name="c", subcore_axis_name="s")

    @functools.partial(
        pl.kernel,
        out_type=(
            jax.ShapeDtypeStruct((nd_pad, dext), F32),
            jax.ShapeDtypeStruct((nd_pad, dext), F32),
        ),
        mesh=mesh,
        scratch_types=[
            pltpu.VMEM((k,), I32), pltpu.VMEM((k,), I32),
            pltpu.VMEM((k,), I32), pltpu.VMEM((k,), I32),
            pltpu.VMEM((k, dext), F32), pltpu.VMEM((k, dext), F32),
            pltpu.VMEM((k, dext), F32), pltpu.VMEM((k, dext), F32),
            pltpu.SemaphoreType.DMA, pltpu.SemaphoreType.DMA,
            pltpu.SemaphoreType.DMA, pltpu.SemaphoreType.DMA,
            pltpu.SemaphoreType.DMA, pltpu.SemaphoreType.DMA,
            pltpu.SemaphoreType.DMA, pltpu.SemaphoreType.DMA,
            pltpu.SemaphoreType.DMA, pltpu.SemaphoreType.DMA,
            pltpu.SemaphoreType.DMA, pltpu.SemaphoreType.DMA,
            pltpu.VMEM_SHARED((nd_pad, dext), F32),
        ],
    )
    def sk(e0_hbm, e1_hbm, dst_hbm, z_hbm, out0, out1,
           idx0, idx1, idx2, idx3, rows0, rows1, rows2, rows3,
           semi0, semi1, semi2, semi3, seml0, seml1, seml2, seml3,
           sems0, sems1, sems2, sems3, acc):
        cid = lax.axis_index("c")
        sid = lax.axis_index("s")
        my_rows = pl.ds(sid * rows_per_tile, rows_per_tile)
        idxv = (idx0, idx1, idx2, idx3)
        rows = (rows0, rows1, rows2, rows3)
        semi = (semi0, semi1, semi2, semi3)
        seml = (seml0, seml1, seml2, seml3)
        sems = (sems0, sems1, sems2, sems3)

        pltpu.sync_copy(z_hbm, acc.at[my_rows])
        plsc.subcore_barrier()

        base = sid * per_tile
        assert nblk % 4 == 2

        def body(ext_hbm):
            def load(j, s):
                pltpu.async_copy(dst_hbm.at[pl.ds(base + j * k, k)], idxv[s],
                                 semi[s])
                pltpu.async_copy(ext_hbm.at[pl.ds(base + j * k, k)], rows[s],
                                 seml[s])

            def wait_load(s):
                pltpu.make_async_copy(dst_hbm.at[pl.ds(base, k)], idxv[s],
                                      semi[s]).wait()
                pltpu.make_async_copy(ext_hbm.at[pl.ds(base, k)], rows[s],
                                      seml[s]).wait()

            def scat(j, s):
                pltpu.async_copy(rows[s], acc.at[idxv[s]], sems[s], add=True)

            def wait_scat(s):
                pltpu.make_async_copy(ext_hbm.at[pl.ds(base, k)], rows[s],
                                      sems[s]).wait()

            load(0, 0)
            load(1, 1)
            for j in (0, 1, 2, 3):
                wait_load(j)
                scat(j, j)
                if j >= 2:
                    wait_scat((j + 2) % 4)
                load(j + 2, (j + 2) % 4)

            def step(j4, carry):
                for s in (0, 1, 2, 3):
                    j = j4 * 4 + s
                    wait_load(s)
                    scat(j, s)
                    wait_scat((s + 2) % 4)
                    load(j + 2, (s + 2) % 4)
                return carry

            lax.fori_loop(1, nblk // 4, step, 0)
            for t, s in ((nblk - 2, (nblk - 2) % 4), (nblk - 1, (nblk - 1) % 4)):
                wait_load(s)
                scat(t, s)
            for s in (0, 1, 2, 3):
                wait_scat(s)

        @pl.when(cid == 0)
        def _():
            body(e0_hbm)

        @pl.when(cid == 1)
        def _():
            body(e1_hbm)

        plsc.subcore_barrier()

        @pl.when(cid == 0)
        def _():
            pltpu.sync_copy(acc.at[my_rows], out0.at[my_rows])

        @pl.when(cid == 1)
        def _():
            pltpu.sync_copy(acc.at[my_rows], out1.at[my_rows])

    return sk(ext0, ext1, dst_idx, zeros)


def _tc_edge_pass(src_data, qe_raw, edge_feat, dt2, wq_parts, wk_parts, wv_parts,
                  twp, tbp, e, nd, b, dh, dext):
    wqn_t, bqp = wq_parts
    wkn_t, wke_t, wkt_t, bk2 = wk_parts
    wvn_t, wve_t, wvt_t, bv2 = wv_parts
    row_off = nd // b
    de = edge_feat.shape[1]

    _C = (1.0, -19.739208221435547, 64.93939208984375, -85.4566879272461,
          60.24246597290039, -26.406761169433594, 7.8066086769104,
          -1.4609479904174805)

    def body(kv_ref, qe_ref, ef_ref, dt_ref,
             wqn, bqr, wkn, wke, wkt, bkr, wvn, wve, wvt, bvr, twr, tbr,
             o0_ref, o1_ref):
        u = dt_ref[...] * twr[...] + tbr[...]
        fr = u - jnp.floor(u + 0.5)
        t = fr * fr
        tf = _C[7]
        for c in (_C[6], _C[5], _C[4], _C[3], _C[2], _C[1], _C[0]):
            tf = tf * t + c
        kv = kv_ref[...]
        ef = ef_ref[...]
        q_mat = jnp.dot(qe_ref[...], wqn[...], preferred_element_type=F32) + bqr[...]
        k_mat = (
            jnp.dot(kv, wkn[...], preferred_element_type=F32)
            + jnp.dot(ef, wke[...], preferred_element_type=F32)
            + jnp.dot(tf, wkt[...], preferred_element_type=F32)
            + bkr[...]
        )
        v_mat = (
            jnp.dot(kv, wvn[...], preferred_element_type=F32)
            + jnp.dot(ef, wve[...], preferred_element_type=F32)
            + jnp.dot(tf, wvt[...], preferred_element_type=F32)
            + bvr[...]
        )
        s = q_mat * k_mat
        ones_h = jnp.ones((dh, 1), F32)
        d0 = jnp.dot(s[:, :dh], ones_h, preferred_element_type=F32)
        d1 = jnp.dot(s[:, dh:], ones_h, preferred_element_type=F32)
        l0 = jnp.where(d0 >= 0, d0, 0.2 * d0)
        l1 = jnp.where(d1 >= 0, d1, 0.2 * d1)
        ex0 = jnp.exp(jnp.minimum(l0, 75.0))
        ex1 = jnp.exp(jnp.minimum(l1, 75.0))
        z = jnp.zeros((b, dext - dh - 1), F32)
        o0_ref[...] = jnp.concatenate(
            [v_mat[:, :dh] * jnp.broadcast_to(ex0, (b, dh)), ex0, z], axis=1
        )
        o1_ref[...] = jnp.concatenate(
            [v_mat[:, dh:] * jnp.broadcast_to(ex1, (b, dh)), ex1, z], axis=1
        )

    wspec = lambda shape: pl.BlockSpec(shape, lambda i: (0, 0))
    return pl.pallas_call(
        body,
        grid=(e // b,),
        in_specs=[
            pl.BlockSpec((b, 128), lambda i: (i + row_off, 0)),
            pl.BlockSpec((b, 128), lambda i: (i, 0)),
            pl.BlockSpec((b, de), lambda i: (i, 0)),
            pl.BlockSpec((b, 1), lambda i: (i, 0)),
            wspec((128, 128)), wspec((1, 128)),
            wspec((128, 128)), wspec((de, 128)), wspec((128, 128)), wspec((1, 128)),
            wspec((128, 128)), wspec((de, 128)), wspec((128, 128)), wspec((1, 128)),
            wspec((1, 128)), wspec((1, 128)),
        ],
        out_specs=(
            pl.BlockSpec((b, dext), lambda i: (i, 0)),
            pl.BlockSpec((b, dext), lambda i: (i, 0)),
        ),
        out_shape=(
            jax.ShapeDtypeStruct((e, dext), F32),
            jax.ShapeDtypeStruct((e, dext), F32),
        ),
        compiler_params=pltpu.CompilerParams(
            dimension_semantics=("parallel",)
        ),
    )(src_data, qe_raw, edge_feat, dt2, wqn_t, bqp,
      wkn_t, wke_t, wkt_t, bk2, wvn_t, wve_t, wvt_t, bv2, twp, tbp)


def _tc_final(p0, p1, src_data, wod_t, wos_t, bo2, g2, b2, nd, b, dh, dext):
    def body(p0_ref, p1_ref, q_ref, wod, wos, bor, gr, br, o_ref):
        p0 = p0_ref[...]
        p1 = p1_ref[...]
        den0 = jnp.maximum(p0[:, dh : dh + 1], 1e-16)
        den1 = jnp.maximum(p1[:, dh : dh + 1], 1e-16)
        dst_h = jnp.concatenate(
            [p0[:, :dh] / jnp.broadcast_to(den0, (b, dh)),
             p1[:, :dh] / jnp.broadcast_to(den1, (b, dh))], axis=1
        )
        r = (
            jnp.dot(dst_h, wod[...], preferred_element_type=F32)
            + jnp.dot(q_ref[...], wos[...], preferred_element_type=F32)
            + bor[...]
        )
        r = jnp.maximum(r, 0.0)
        mu = jnp.mean(r, axis=1, keepdims=True)
        var = jnp.mean((r - mu) ** 2, axis=1, keepdims=True)
        o_ref[...] = (r - mu) / jnp.sqrt(var + 1e-5) * gr[...] + br[...]

    wspec = lambda shape: pl.BlockSpec(shape, lambda i: (0, 0))
    return pl.pallas_call(
        body,
        grid=(nd // b,),
        in_specs=[
            pl.BlockSpec((b, dext), lambda i: (i, 0)),
            pl.BlockSpec((b, dext), lambda i: (i, 0)),
            pl.BlockSpec((b, 128), lambda i: (i, 0)),
            wspec((128, 128)), wspec((128, 128)),
            wspec((1, 128)), wspec((1, 128)), wspec((1, 128)),
        ],
        out_specs=pl.BlockSpec((b, 128), lambda i: (i, 0)),
        out_shape=jax.ShapeDtypeStruct((nd, 128), F32),
        compiler_params=pltpu.CompilerParams(
            dimension_semantics=("parallel",)
        ),
    )(p0, p1, src_data, wod_t, wos_t, bo2, g2, b2)


def kernel(h, src_idx, edge_dt, edge_feat, edge_dst, num_dst, time_w, time_b,
           Wq, bq, Wk, bk, Wv, bv, Wo, bo, ln_g, ln_b):
    e = edge_dst.shape[0]
    n_src, dn = h.shape
    nd = n_src - e
    dt_dim = time_w.shape[0]
    dout = Wq.shape[0]
    dh = dout // 2
    de = edge_feat.shape[1]
    dext = 128
    b = 400
    k = 80
    nw = 32

    wqn_t = Wq[:, :dn].T
    bqp = (bq + jnp.cos(time_b) @ Wq[:, dn:].T).reshape(1, dout)
    wkn_t = Wk[:, :dn].T
    wke_t = Wk[:, dn : dn + de].T
    wkt_t = jnp.pad(Wk[:, dn + de :].T, ((0, 128 - dt_dim), (0, 0)))
    wvn_t = Wv[:, :dn].T
    wve_t = Wv[:, dn : dn + de].T
    wvt_t = jnp.pad(Wv[:, dn + de :].T, ((0, 128 - dt_dim), (0, 0)))
    inv2pi = 1.0 / (2.0 * jnp.pi)
    twp = (jnp.pad(time_w[:, 0], (0, 128 - dt_dim)) * inv2pi).reshape(1, 128)
    tbp = (jnp.pad(time_b, (0, 128 - dt_dim)) * inv2pi).reshape(1, 128)
    wod_t = Wo[:, :dout].T
    wos_t = Wo[:, dout:].T
    bo2 = bo.reshape(1, dout)
    g2 = ln_g.reshape(1, dout)
    b2 = ln_b.reshape(1, dout)
    bk2 = bk.reshape(1, dout)
    bv2 = bv.reshape(1, dout)

    k0 = 120
    chunk = nw * k0 * 2
    n_pad = ((n_src + chunk - 1) // chunk) * chunk
    si = jnp.pad(src_idx.astype(I32), (0, n_pad - n_src))
    src_data = _sc_gather(h, si, k0)

    dst_i = edge_dst.astype(I32)
    k2 = 128
    chunk2 = nw * k2 * 2
    e_pad = ((e + chunk2 - 1) // chunk2) * chunk2
    dst_pad = jnp.pad(dst_i, (0, e_pad - e))
    qe_raw = _sc_gather_small(src_data, dst_pad, k2, 10240)

    b3 = 2000
    dt2 = edge_dt.reshape(e, 1)
    ext0, ext1 = _tc_edge_pass(src_data, qe_raw, edge_feat, dt2,
                               (wqn_t, bqp),
                               (wkn_t, wke_t, wkt_t, bk2), (wvn_t, wve_t, wvt_t, bv2),
                               twp, tbp, e, nd, b3, dh, dext)

    p0, p1 = _sc_scatter_add(ext0, ext1, dst_i, nd, k)

    return _tc_final(p0, p1, src_data, wod_t, wos_t, bo2, g2, b2, nd, b, dh, dext)
